# baseline (speedup 1.0000x reference)
import functools

import jax
import jax.numpy as jnp
from jax import lax
from jax.experimental import pallas as pl
from jax.experimental.pallas import tpu as pltpu
from jax.experimental.pallas import tpu_sc as plsc

N = 10000
E = 320000
FN = 128
FE = 16
H = 64

NC = 2
NS = 16
NW = NC * NS
CHUNK = 128
NBUF = 4
K = (-(-E // (NW * CHUNK)) + NBUF - 1) // NBUF * NBUF
E_PAD = NW * K * CHUNK
N_ACC = 10240
RPT = N_ACC // NS
DUMMY = N


def _sc_mesh():
    return plsc.VectorSubcoreMesh(core_axis_name="c", subcore_axis_name="s",
                                  num_cores=NC, num_subcores=NS)


_SC_PARAMS = pltpu.CompilerParams(use_tc_tiling_on_sc=False)


def _zero_rows(ref, nrows, ncols):
    def body(i, _):
        for cth in range(ncols // 16):
            ref[i, pl.ds(cth * 16, 16)] = jnp.zeros((16,), jnp.float32)
        return 0
    lax.fori_loop(0, nrows, body, 0)


@functools.lru_cache(maxsize=None)
def _build_sc():
    interpret = False

    @functools.partial(
        pl.kernel,
        out_type=jax.ShapeDtypeStruct((NC, N_ACC, 16), jnp.float32),
        mesh=_sc_mesh(),
        scratch_types=[
            pltpu.VMEM((K, CHUNK), jnp.int32),
            pltpu.VMEM((CHUNK, 16), jnp.float32),
            pltpu.VMEM((RPT, 16), jnp.float32),
            pltpu.VMEM_SHARED((N_ACC, 16), jnp.float32),
            pltpu.SemaphoreType.DMA,
        ],
        compiler_params=_SC_PARAMS,
        interpret=interpret,
    )
    def deg_kernel(dst_hbm, out_hbm, dst_v, ones_v, stage_v, accum, sem):
        c = lax.axis_index("c")
        s = lax.axis_index("s")
        w = c * NS + s
        base = s * RPT

        def fill_ones(i, _):
            ones_v[i, :] = jnp.ones((16,), jnp.float32)
            return 0
        lax.fori_loop(0, CHUNK, fill_ones, 0)
        _zero_rows(stage_v, RPT, 16)

        pltpu.sync_copy(dst_hbm.at[w], dst_v)
        pltpu.sync_copy(stage_v, accum.at[pl.ds(base, RPT)])
        plsc.subcore_barrier()

        def body(j, _):
            pltpu.async_copy(ones_v, accum.at[dst_v.at[j]], sem, add=True)
            return 0
        lax.fori_loop(0, K, body, 0)

        def drain(j, _):
            pltpu.make_async_copy(ones_v, accum.at[dst_v.at[j]], sem).wait()
            return 0
        lax.fori_loop(0, K, drain, 0)

        plsc.subcore_barrier()
        pltpu.sync_copy(accum.at[pl.ds(base, RPT)], stage_v)
        pltpu.sync_copy(stage_v, out_hbm.at[c, pl.ds(base, RPT)])

    @functools.partial(
        pl.kernel,
        out_type=jax.ShapeDtypeStruct((NC, N_ACC, H), jnp.float32),
        mesh=_sc_mesh(),
        scratch_types=[
            pltpu.VMEM((K, CHUNK), jnp.int32),
            pltpu.VMEM((K, CHUNK), jnp.int32),
            pltpu.VMEM((NBUF, CHUNK, H), jnp.float32),
            pltpu.VMEM((CHUNK, H), jnp.float32),
            pltpu.VMEM_SHARED((N_ACC, H), jnp.float32),
            pltpu.SemaphoreType.DMA((NBUF,)),
            pltpu.SemaphoreType.DMA((NBUF,)),
        ],
        compiler_params=_SC_PARAMS,
        interpret=interpret,
    )
    def scatter_kernel(y_hbm, src_hbm, dst_hbm, out_hbm,
                       src_v, dst_v, rows_v, stage_v, accum, gsem, ssem):
        c = lax.axis_index("c")
        s = lax.axis_index("s")
        w = c * NS + s
        base = s * RPT

        _zero_rows(stage_v, CHUNK, H)
        pltpu.sync_copy(src_hbm.at[w], src_v)
        pltpu.sync_copy(dst_hbm.at[w], dst_v)
        for r in range(RPT // CHUNK):
            pltpu.sync_copy(stage_v, accum.at[pl.ds(base + r * CHUNK, CHUNK)])
        plsc.subcore_barrier()

        for b in range(NBUF):
            pltpu.async_copy(y_hbm.at[src_v.at[b]], rows_v.at[b], gsem.at[b])

        def body(jj, _):
            for b in range(NBUF):
                j = jj + b
                pltpu.make_async_copy(
                    y_hbm.at[src_v.at[j]], rows_v.at[b], gsem.at[b]).wait()
                pltpu.async_copy(
                    rows_v.at[b], accum.at[dst_v.at[j]], ssem.at[b], add=True)
            for b in range(NBUF):
                j = jj + b
                pltpu.make_async_copy(
                    rows_v.at[b], accum.at[dst_v.at[j]], ssem.at[b]).wait()

                @pl.when(j + NBUF < K)
                def _():
                    pltpu.async_copy(
                        y_hbm.at[src_v.at[j + NBUF]], rows_v.at[b], gsem.at[b])
            return 0
        lax.fori_loop(0, K // NBUF, lambda i, cc: body(i * NBUF, cc), 0)

        plsc.subcore_barrier()
        for r in range(RPT // CHUNK):
            pltpu.sync_copy(accum.at[pl.ds(base + r * CHUNK, CHUNK)], stage_v)
            pltpu.sync_copy(stage_v, out_hbm.at[c, pl.ds(base + r * CHUNK, CHUNK)])

    @functools.partial(
        pl.kernel,
        out_type=(jax.ShapeDtypeStruct((E_PAD, H), jnp.float32),
                  jax.ShapeDtypeStruct((E_PAD, H), jnp.float32)),
        mesh=_sc_mesh(),
        scratch_types=[
            pltpu.VMEM((K, CHUNK), jnp.int32),
            pltpu.VMEM((K, CHUNK), jnp.int32),
            pltpu.VMEM((NBUF, CHUNK, H), jnp.float32),
            pltpu.VMEM((NBUF, CHUNK, H), jnp.float32),
            pltpu.SemaphoreType.DMA((NBUF,)),
            pltpu.SemaphoreType.DMA((NBUF,)),
            pltpu.SemaphoreType.DMA((NBUF,)),
            pltpu.SemaphoreType.DMA((NBUF,)),
        ],
        compiler_params=_SC_PARAMS,
        interpret=interpret,
    )
    def gather_kernel(p_hbm, q_hbm, src_hbm, dst_hbm, outp_hbm, outq_hbm,
                      src_v, dst_v, bufp_v, bufq_v, gpsem, gqsem, wpsem, wqsem):
        c = lax.axis_index("c")
        s = lax.axis_index("s")
        w = c * NS + s
        ebase = w * K * CHUNK

        pltpu.sync_copy(src_hbm.at[w], src_v)
        pltpu.sync_copy(dst_hbm.at[w], dst_v)

        for b in range(NBUF):
            pltpu.async_copy(p_hbm.at[src_v.at[b]], bufp_v.at[b], gpsem.at[b])
            pltpu.async_copy(q_hbm.at[dst_v.at[b]], bufq_v.at[b], gqsem.at[b])

        def body(jj, _):
            for b in range(NBUF):
                j = jj + b
                off = ebase + j * CHUNK
                pltpu.make_async_copy(
                    p_hbm.at[src_v.at[j]], bufp_v.at[b], gpsem.at[b]).wait()
                pltpu.async_copy(
                    bufp_v.at[b], outp_hbm.at[pl.ds(off, CHUNK)], wpsem.at[b])
                pltpu.make_async_copy(
                    q_hbm.at[dst_v.at[j]], bufq_v.at[b], gqsem.at[b]).wait()
                pltpu.async_copy(
                    bufq_v.at[b], outq_hbm.at[pl.ds(off, CHUNK)], wqsem.at[b])
            for b in range(NBUF):
                j = jj + b
                off = ebase + j * CHUNK
                pltpu.make_async_copy(
                    bufp_v.at[b], outp_hbm.at[pl.ds(off, CHUNK)], wpsem.at[b]).wait()
                pltpu.make_async_copy(
                    bufq_v.at[b], outq_hbm.at[pl.ds(off, CHUNK)], wqsem.at[b]).wait()

                @pl.when(j + NBUF < K)
                def _():
                    pltpu.async_copy(
                        p_hbm.at[src_v.at[j + NBUF]], bufp_v.at[b], gpsem.at[b])
                    pltpu.async_copy(
                        q_hbm.at[dst_v.at[j + NBUF]], bufq_v.at[b], gqsem.at[b])
            return 0
        lax.fori_loop(0, K // NBUF, lambda i, cc: body(i * NBUF, cc), 0)

    return dict(deg=deg_kernel, scatter=scatter_kernel, gather=gather_kernel)


@functools.lru_cache(maxsize=None)
def _build_tc(interpret: bool = False):

    def _mm(a, b):
        return jnp.dot(a, b, preferred_element_type=jnp.float32)

    BLK_N = 1000
    BLK_E = 2000

    def tc_a_body(x_ref, w1_ref, d0_ref, d1_ref, y_ref, dinv_ref):
        dinv = lax.rsqrt(d0_ref[...] + d1_ref[...] + 1.0)
        y_ref[...] = _mm(x_ref[...], w1_ref[...]) * dinv
        dinv_ref[...] = dinv

    tc_a = pl.pallas_call(
        tc_a_body,
        grid=(N // BLK_N,),
        in_specs=[
            pl.BlockSpec((BLK_N, FN), lambda i: (i, 0)),
            pl.BlockSpec((FN, H), lambda i: (0, 0)),
            pl.BlockSpec((BLK_N, 1), lambda i: (i, 0)),
            pl.BlockSpec((BLK_N, 1), lambda i: (i, 0)),
        ],
        out_specs=[
            pl.BlockSpec((BLK_N, H), lambda i: (i, 0)),
            pl.BlockSpec((BLK_N, 1), lambda i: (i, 0)),
        ],
        out_shape=[
            jax.ShapeDtypeStruct((N, H), jnp.float32),
            jax.ShapeDtypeStruct((N, 1), jnp.float32),
        ],
        interpret=interpret,
    )

    def tc_layer_body(a0_ref, a1_ref, yp_ref, dinv_ref, b_ref, w_ref, y_ref):
        h = jnp.maximum(
            (a0_ref[...] + a1_ref[...] + yp_ref[...]) * dinv_ref[...] + b_ref[...],
            0.0)
        y_ref[...] = _mm(h, w_ref[...]) * dinv_ref[...]

    tc_layer = pl.pallas_call(
        tc_layer_body,
        grid=(N // BLK_N,),
        in_specs=[
            pl.BlockSpec((BLK_N, H), lambda i: (i, 0)),
            pl.BlockSpec((BLK_N, H), lambda i: (i, 0)),
            pl.BlockSpec((BLK_N, H), lambda i: (i, 0)),
            pl.BlockSpec((BLK_N, 1), lambda i: (i, 0)),
            pl.BlockSpec((1, H), lambda i: (0, 0)),
            pl.BlockSpec((H, H), lambda i: (0, 0)),
        ],
        out_specs=pl.BlockSpec((BLK_N, H), lambda i: (i, 0)),
        out_shape=jax.ShapeDtypeStruct((N, H), jnp.float32),
        interpret=interpret,
    )

    def tc_final_body(a0_ref, a1_ref, yp_ref, dinv_ref, b_ref, wa_ref, wb_ref,
                      p_ref, q_ref):
        h = jnp.maximum(
            (a0_ref[...] + a1_ref[...] + yp_ref[...]) * dinv_ref[...] + b_ref[...],
            0.0)
        p_ref[...] = _mm(h, wa_ref[...])
        q_ref[...] = _mm(h, wb_ref[...])

    tc_final = pl.pallas_call(
        tc_final_body,
        grid=(N // BLK_N,),
        in_specs=[
            pl.BlockSpec((BLK_N, H), lambda i: (i, 0)),
            pl.BlockSpec((BLK_N, H), lambda i: (i, 0)),
            pl.BlockSpec((BLK_N, H), lambda i: (i, 0)),
            pl.BlockSpec((BLK_N, 1), lambda i: (i, 0)),
            pl.BlockSpec((1, H), lambda i: (0, 0)),
            pl.BlockSpec((H, H), lambda i: (0, 0)),
            pl.BlockSpec((H, H), lambda i: (0, 0)),
        ],
        out_specs=[
            pl.BlockSpec((BLK_N, H), lambda i: (i, 0)),
            pl.BlockSpec((BLK_N, H), lambda i: (i, 0)),
        ],
        out_shape=[
            jax.ShapeDtypeStruct((N, H), jnp.float32),
            jax.ShapeDtypeStruct((N, H), jnp.float32),
        ],
        interpret=interpret,
    )

    def tc_edge_body(ps_ref, qd_ref, ea_ref, we1_ref, be1_ref, we2_ref, be2_ref,
                     wc1c_ref, bc1_ref, wc2_ref, bc2_ref, wc3_ref, bc3_ref,
                     out_ref):
        e1 = jnp.maximum(_mm(ea_ref[...], we1_ref[...]) + be1_ref[...], 0.0)
        wfold = _mm(we2_ref[...], wc1c_ref[...])
        bfold = _mm(be2_ref[...], wc1c_ref[...]) + bc1_ref[...]
        g = _mm(e1, wfold) + bfold
        z1 = jnp.maximum(ps_ref[...] + qd_ref[...] + g, 0.0)
        z2 = jnp.maximum(_mm(z1, wc2_ref[...]) + bc2_ref[...], 0.0)
        z3 = _mm(z2, wc3_ref[...]) + bc3_ref[...]
        m = jnp.max(z3, axis=1, keepdims=True)
        lse = m + jnp.log(jnp.sum(jnp.exp(z3 - m), axis=1, keepdims=True))
        out_ref[...] = z3 - lse

    tc_edge = pl.pallas_call(
        tc_edge_body,
        grid=(E // BLK_E,),
        in_specs=[
            pl.BlockSpec((BLK_E, H), lambda i: (i, 0)),
            pl.BlockSpec((BLK_E, H), lambda i: (i, 0)),
            pl.BlockSpec((BLK_E, FE), lambda i: (i, 0)),
            pl.BlockSpec((FE, H), lambda i: (0, 0)),
            pl.BlockSpec((1, H), lambda i: (0, 0)),
            pl.BlockSpec((H, H), lambda i: (0, 0)),
            pl.BlockSpec((1, H), lambda i: (0, 0)),
            pl.BlockSpec((H, H), lambda i: (0, 0)),
            pl.BlockSpec((1, H), lambda i: (0, 0)),
            pl.BlockSpec((H, H // 2), lambda i: (0, 0)),
            pl.BlockSpec((1, H // 2), lambda i: (0, 0)),
            pl.BlockSpec((H // 2, 2), lambda i: (0, 0)),
            pl.BlockSpec((1, 2), lambda i: (0, 0)),
        ],
        out_specs=pl.BlockSpec((BLK_E, 2), lambda i: (i, 0)),
        out_shape=jax.ShapeDtypeStruct((E, 2), jnp.float32),
        interpret=interpret,
    )

    return dict(tc_a=tc_a, tc_layer=tc_layer, tc_final=tc_final,
                tc_edge=tc_edge)


def kernel(x, edge_index, edge_attr, W1, b1, W2, b2, W3, b3,
           We1, be1, We2, be2, Wc1, bc1, Wc2, bc2, Wc3, bc3):
    k = dict(_build_sc())
    k.update(_build_tc(False))
    ei = edge_index.astype(jnp.int32)
    src, dst = ei[0], ei[1]
    pad = E_PAD - E
    src_p = jnp.concatenate([src, jnp.zeros((pad,), jnp.int32)]).reshape(NW, K, CHUNK)
    dst_p = jnp.concatenate([dst, jnp.full((pad,), DUMMY, jnp.int32)]).reshape(NW, K, CHUNK)
    dst_g = jnp.concatenate([dst, jnp.zeros((pad,), jnp.int32)]).reshape(NW, K, CHUNK)

    degp = k["deg"](dst_p)
    d0 = degp[0, :N, 0:1]
    d1 = degp[1, :N, 0:1]

    y1, dinv = k["tc_a"](x, W1, d0, d1)
    agg = k["scatter"](y1, src_p, dst_p)
    y2 = k["tc_layer"](agg[0, :N], agg[1, :N], y1, dinv, b1.reshape(1, H), W2)
    agg = k["scatter"](y2, src_p, dst_p)
    y3 = k["tc_layer"](agg[0, :N], agg[1, :N], y2, dinv, b2.reshape(1, H), W3)
    agg = k["scatter"](y3, src_p, dst_p)
    P, Q = k["tc_final"](agg[0, :N], agg[1, :N], y3, dinv, b3.reshape(1, H),
                         Wc1[:H], Wc1[H:2 * H])
    Ps, Qd = k["gather"](P, Q, src_p, dst_g)
    out = k["tc_edge"](Ps[:E], Qd[:E], edge_attr,
                       We1, be1.reshape(1, H), We2, be2.reshape(1, H),
                       Wc1[2 * H:], bc1.reshape(1, H),
                       Wc2, bc2.reshape(1, H // 2), Wc3, bc3.reshape(1, 2))
    return out


# speedup vs baseline: 1.5206x; 1.5206x over previous
import functools

import jax
import jax.numpy as jnp
from jax import lax
from jax.experimental import pallas as pl
from jax.experimental.pallas import tpu as pltpu
from jax.experimental.pallas import tpu_sc as plsc

N = 10000
E = 320000
FN = 128
FE = 16
H = 64

NC = 2
NS = 16
NW = NC * NS
CHUNK = 128
NBUF = 4
K = (-(-E // (NW * CHUNK)) + NBUF - 1) // NBUF * NBUF
E_PAD = NW * K * CHUNK
N_ACC = 10240
RPT = N_ACC // NS
DUMMY = N


def _sc_mesh():
    return plsc.VectorSubcoreMesh(core_axis_name="c", subcore_axis_name="s",
                                  num_cores=NC, num_subcores=NS)


_SC_PARAMS = pltpu.CompilerParams(use_tc_tiling_on_sc=False)


def _zero_rows(ref, nrows, ncols):
    def body(i, _):
        for cth in range(ncols // 16):
            ref[i, pl.ds(cth * 16, 16)] = jnp.zeros((16,), jnp.float32)
        return 0
    lax.fori_loop(0, nrows, body, 0)


@functools.lru_cache(maxsize=None)
def _build_sc():
    interpret = False

    @functools.partial(
        pl.kernel,
        out_type=jax.ShapeDtypeStruct((NC, N_ACC, 16), jnp.float32),
        mesh=_sc_mesh(),
        scratch_types=[
            pltpu.VMEM((K, CHUNK), jnp.int32),
            pltpu.VMEM((CHUNK, 16), jnp.float32),
            pltpu.VMEM((RPT, 16), jnp.float32),
            pltpu.VMEM_SHARED((N_ACC, 16), jnp.float32),
            pltpu.SemaphoreType.DMA,
        ],
        compiler_params=_SC_PARAMS,
        interpret=interpret,
    )
    def deg_kernel(dst_hbm, out_hbm, dst_v, ones_v, stage_v, accum, sem):
        c = lax.axis_index("c")
        s = lax.axis_index("s")
        w = c * NS + s
        base = s * RPT

        def fill_ones(i, _):
            ones_v[i, :] = jnp.ones((16,), jnp.float32)
            return 0
        lax.fori_loop(0, CHUNK, fill_ones, 0)
        _zero_rows(stage_v, RPT, 16)

        pltpu.sync_copy(dst_hbm.at[w], dst_v)
        pltpu.sync_copy(stage_v, accum.at[pl.ds(base, RPT)])
        plsc.subcore_barrier()

        def body(j, _):
            pltpu.async_copy(ones_v, accum.at[dst_v.at[j]], sem, add=True)
            return 0
        lax.fori_loop(0, K, body, 0)

        def drain(j, _):
            pltpu.make_async_copy(ones_v, accum.at[dst_v.at[j]], sem).wait()
            return 0
        lax.fori_loop(0, K, drain, 0)

        plsc.subcore_barrier()
        pltpu.sync_copy(accum.at[pl.ds(base, RPT)], stage_v)
        pltpu.sync_copy(stage_v, out_hbm.at[c, pl.ds(base, RPT)])

    @functools.partial(
        pl.kernel,
        out_type=jax.ShapeDtypeStruct((NC, N_ACC, H), jnp.float32),
        mesh=_sc_mesh(),
        scratch_types=[
            pltpu.VMEM((K, CHUNK), jnp.int32),
            pltpu.VMEM((K, CHUNK), jnp.int32),
            pltpu.VMEM((NBUF, CHUNK, H), jnp.float32),
            pltpu.VMEM((CHUNK, H), jnp.float32),
            pltpu.VMEM_SHARED((N_ACC, H), jnp.float32),
            pltpu.SemaphoreType.DMA((NBUF,)),
            pltpu.SemaphoreType.DMA((NBUF,)),
        ],
        compiler_params=_SC_PARAMS,
        interpret=interpret,
    )
    def scatter_kernel(y_hbm, src_hbm, dst_hbm, out_hbm,
                       src_v, dst_v, rows_v, stage_v, accum, gsem, ssem):
        c = lax.axis_index("c")
        s = lax.axis_index("s")
        w = c * NS + s
        base = s * RPT

        _zero_rows(stage_v, CHUNK, H)
        pltpu.sync_copy(src_hbm.at[w], src_v)
        pltpu.sync_copy(dst_hbm.at[w], dst_v)
        for r in range(RPT // CHUNK):
            pltpu.sync_copy(stage_v, accum.at[pl.ds(base + r * CHUNK, CHUNK)])
        plsc.subcore_barrier()

        for b in range(NBUF):
            pltpu.async_copy(y_hbm.at[src_v.at[b]], rows_v.at[b], gsem.at[b])

        def body(jj, _):
            for b in range(NBUF):
                j = jj + b
                pltpu.make_async_copy(
                    y_hbm.at[src_v.at[j]], rows_v.at[b], gsem.at[b]).wait()
                pltpu.async_copy(
                    rows_v.at[b], accum.at[dst_v.at[j]], ssem.at[b], add=True)
            for b in range(NBUF):
                j = jj + b
                pltpu.make_async_copy(
                    rows_v.at[b], accum.at[dst_v.at[j]], ssem.at[b]).wait()

                @pl.when(j + NBUF < K)
                def _():
                    pltpu.async_copy(
                        y_hbm.at[src_v.at[j + NBUF]], rows_v.at[b], gsem.at[b])
            return 0
        lax.fori_loop(0, K // NBUF, lambda i, cc: body(i * NBUF, cc), 0)

        plsc.subcore_barrier()
        for r in range(RPT // CHUNK):
            pltpu.sync_copy(accum.at[pl.ds(base + r * CHUNK, CHUNK)], stage_v)
            pltpu.sync_copy(stage_v, out_hbm.at[c, pl.ds(base + r * CHUNK, CHUNK)])

    @functools.partial(
        pl.kernel,
        out_type=(jax.ShapeDtypeStruct((E_PAD, H), jnp.float32),
                  jax.ShapeDtypeStruct((E_PAD, H), jnp.float32)),
        mesh=_sc_mesh(),
        scratch_types=[
            pltpu.VMEM((K, CHUNK), jnp.int32),
            pltpu.VMEM((K, CHUNK), jnp.int32),
            pltpu.VMEM((NBUF, CHUNK, H), jnp.float32),
            pltpu.VMEM((NBUF, CHUNK, H), jnp.float32),
            pltpu.SemaphoreType.DMA((NBUF,)),
            pltpu.SemaphoreType.DMA((NBUF,)),
            pltpu.SemaphoreType.DMA((NBUF,)),
            pltpu.SemaphoreType.DMA((NBUF,)),
        ],
        compiler_params=_SC_PARAMS,
        interpret=interpret,
    )
    def gather_kernel(p_hbm, q_hbm, src_hbm, dst_hbm, outp_hbm, outq_hbm,
                      src_v, dst_v, bufp_v, bufq_v, gpsem, gqsem, wpsem, wqsem):
        c = lax.axis_index("c")
        s = lax.axis_index("s")
        w = c * NS + s

        pltpu.sync_copy(src_hbm.at[w], src_v)
        pltpu.sync_copy(dst_hbm.at[w], dst_v)

        for b in range(NBUF):
            pltpu.async_copy(p_hbm.at[src_v.at[b]], bufp_v.at[b], gpsem.at[b])
            pltpu.async_copy(q_hbm.at[dst_v.at[b]], bufq_v.at[b], gqsem.at[b])

        def body(jj, _):
            for b in range(NBUF):
                j = jj + b
                off = (j * NW + w) * CHUNK
                pltpu.make_async_copy(
                    p_hbm.at[src_v.at[j]], bufp_v.at[b], gpsem.at[b]).wait()
                pltpu.async_copy(
                    bufp_v.at[b], outp_hbm.at[pl.ds(off, CHUNK)], wpsem.at[b])
                pltpu.make_async_copy(
                    q_hbm.at[dst_v.at[j]], bufq_v.at[b], gqsem.at[b]).wait()
                pltpu.async_copy(
                    bufq_v.at[b], outq_hbm.at[pl.ds(off, CHUNK)], wqsem.at[b])
            for b in range(NBUF):
                j = jj + b
                off = (j * NW + w) * CHUNK
                pltpu.make_async_copy(
                    bufp_v.at[b], outp_hbm.at[pl.ds(off, CHUNK)], wpsem.at[b]).wait()
                pltpu.make_async_copy(
                    bufq_v.at[b], outq_hbm.at[pl.ds(off, CHUNK)], wqsem.at[b]).wait()

                @pl.when(j + NBUF < K)
                def _():
                    pltpu.async_copy(
                        p_hbm.at[src_v.at[j + NBUF]], bufp_v.at[b], gpsem.at[b])
                    pltpu.async_copy(
                        q_hbm.at[dst_v.at[j + NBUF]], bufq_v.at[b], gqsem.at[b])
            return 0
        lax.fori_loop(0, K // NBUF, lambda i, cc: body(i * NBUF, cc), 0)

    return dict(deg=deg_kernel, scatter=scatter_kernel, gather=gather_kernel)


@functools.lru_cache(maxsize=None)
def _build_tc(interpret: bool = False):

    def _mm(a, b):
        return jnp.dot(a, b, preferred_element_type=jnp.float32)

    BLK_N = 1000
    BLK_E = 2000

    def tc_a_body(x_ref, w1_ref, d0_ref, d1_ref, y_ref, dinv_ref):
        dinv = lax.rsqrt(d0_ref[...] + d1_ref[...] + 1.0)
        y_ref[...] = _mm(x_ref[...], w1_ref[...]) * dinv
        dinv_ref[...] = dinv

    tc_a = pl.pallas_call(
        tc_a_body,
        grid=(N // BLK_N,),
        in_specs=[
            pl.BlockSpec((BLK_N, FN), lambda i: (i, 0)),
            pl.BlockSpec((FN, H), lambda i: (0, 0)),
            pl.BlockSpec((BLK_N, 1), lambda i: (i, 0)),
            pl.BlockSpec((BLK_N, 1), lambda i: (i, 0)),
        ],
        out_specs=[
            pl.BlockSpec((BLK_N, H), lambda i: (i, 0)),
            pl.BlockSpec((BLK_N, 1), lambda i: (i, 0)),
        ],
        out_shape=[
            jax.ShapeDtypeStruct((N, H), jnp.float32),
            jax.ShapeDtypeStruct((N, 1), jnp.float32),
        ],
        interpret=interpret,
    )

    def tc_layer_body(a0_ref, a1_ref, yp_ref, dinv_ref, b_ref, w_ref, y_ref):
        h = jnp.maximum(
            (a0_ref[...] + a1_ref[...] + yp_ref[...]) * dinv_ref[...] + b_ref[...],
            0.0)
        y_ref[...] = _mm(h, w_ref[...]) * dinv_ref[...]

    tc_layer = pl.pallas_call(
        tc_layer_body,
        grid=(N // BLK_N,),
        in_specs=[
            pl.BlockSpec((BLK_N, H), lambda i: (i, 0)),
            pl.BlockSpec((BLK_N, H), lambda i: (i, 0)),
            pl.BlockSpec((BLK_N, H), lambda i: (i, 0)),
            pl.BlockSpec((BLK_N, 1), lambda i: (i, 0)),
            pl.BlockSpec((1, H), lambda i: (0, 0)),
            pl.BlockSpec((H, H), lambda i: (0, 0)),
        ],
        out_specs=pl.BlockSpec((BLK_N, H), lambda i: (i, 0)),
        out_shape=jax.ShapeDtypeStruct((N, H), jnp.float32),
        interpret=interpret,
    )

    def tc_final_body(a0_ref, a1_ref, yp_ref, dinv_ref, b_ref, wa_ref, wb_ref,
                      p_ref, q_ref):
        h = jnp.maximum(
            (a0_ref[...] + a1_ref[...] + yp_ref[...]) * dinv_ref[...] + b_ref[...],
            0.0)
        p_ref[...] = _mm(h, wa_ref[...])
        q_ref[...] = _mm(h, wb_ref[...])

    tc_final = pl.pallas_call(
        tc_final_body,
        grid=(N // BLK_N,),
        in_specs=[
            pl.BlockSpec((BLK_N, H), lambda i: (i, 0)),
            pl.BlockSpec((BLK_N, H), lambda i: (i, 0)),
            pl.BlockSpec((BLK_N, H), lambda i: (i, 0)),
            pl.BlockSpec((BLK_N, 1), lambda i: (i, 0)),
            pl.BlockSpec((1, H), lambda i: (0, 0)),
            pl.BlockSpec((H, H), lambda i: (0, 0)),
            pl.BlockSpec((H, H), lambda i: (0, 0)),
        ],
        out_specs=[
            pl.BlockSpec((BLK_N, H), lambda i: (i, 0)),
            pl.BlockSpec((BLK_N, H), lambda i: (i, 0)),
        ],
        out_shape=[
            jax.ShapeDtypeStruct((N, H), jnp.float32),
            jax.ShapeDtypeStruct((N, H), jnp.float32),
        ],
        interpret=interpret,
    )

    def tc_edge_body(ps_ref, qd_ref, ea_ref, we1_ref, be1_ref, we2_ref, be2_ref,
                     wc1c_ref, bc1_ref, wc2_ref, bc2_ref, wc3_ref, bc3_ref,
                     out_ref):
        e1 = jnp.maximum(_mm(ea_ref[...], we1_ref[...]) + be1_ref[...], 0.0)
        wfold = _mm(we2_ref[...], wc1c_ref[...])
        bfold = _mm(be2_ref[...], wc1c_ref[...]) + bc1_ref[...]
        g = _mm(e1, wfold) + bfold
        z1 = jnp.maximum(ps_ref[...] + qd_ref[...] + g, 0.0)
        z2 = jnp.maximum(_mm(z1, wc2_ref[...]) + bc2_ref[...], 0.0)
        z3 = _mm(z2, wc3_ref[...]) + bc3_ref[...]
        m = jnp.max(z3, axis=1, keepdims=True)
        lse = m + jnp.log(jnp.sum(jnp.exp(z3 - m), axis=1, keepdims=True))
        out_ref[...] = z3 - lse

    tc_edge = pl.pallas_call(
        tc_edge_body,
        grid=(E // BLK_E,),
        in_specs=[
            pl.BlockSpec((BLK_E, H), lambda i: (i, 0)),
            pl.BlockSpec((BLK_E, H), lambda i: (i, 0)),
            pl.BlockSpec((BLK_E, FE), lambda i: (i, 0)),
            pl.BlockSpec((FE, H), lambda i: (0, 0)),
            pl.BlockSpec((1, H), lambda i: (0, 0)),
            pl.BlockSpec((H, H), lambda i: (0, 0)),
            pl.BlockSpec((1, H), lambda i: (0, 0)),
            pl.BlockSpec((H, H), lambda i: (0, 0)),
            pl.BlockSpec((1, H), lambda i: (0, 0)),
            pl.BlockSpec((H, H // 2), lambda i: (0, 0)),
            pl.BlockSpec((1, H // 2), lambda i: (0, 0)),
            pl.BlockSpec((H // 2, 2), lambda i: (0, 0)),
            pl.BlockSpec((1, 2), lambda i: (0, 0)),
        ],
        out_specs=pl.BlockSpec((BLK_E, 2), lambda i: (i, 0)),
        out_shape=jax.ShapeDtypeStruct((E, 2), jnp.float32),
        interpret=interpret,
    )

    return dict(tc_a=tc_a, tc_layer=tc_layer, tc_final=tc_final,
                tc_edge=tc_edge)


def kernel(x, edge_index, edge_attr, W1, b1, W2, b2, W3, b3,
           We1, be1, We2, be2, Wc1, bc1, Wc2, bc2, Wc3, bc3):
    k = dict(_build_sc())
    k.update(_build_tc(False))
    ei = edge_index.astype(jnp.int32)
    src, dst = ei[0], ei[1]
    pad = E_PAD - E
    ar = jnp.arange(pad, dtype=jnp.int32)
    pad_lo = ar % jnp.int32(256)
    pad_hi = DUMMY + ar % jnp.int32(N_ACC - N)
    def slab(v):
        return v.reshape(K, NW, CHUNK).transpose(1, 0, 2)
    src_p = slab(jnp.concatenate([src, pad_lo]))
    dst_p = slab(jnp.concatenate([dst, pad_hi]))
    dst_g = slab(jnp.concatenate([dst, pad_lo]))

    degp = k["deg"](dst_p)
    d0 = degp[0, :N, 0:1]
    d1 = degp[1, :N, 0:1]

    y1, dinv = k["tc_a"](x, W1, d0, d1)
    agg = k["scatter"](y1, src_p, dst_p)
    y2 = k["tc_layer"](agg[0, :N], agg[1, :N], y1, dinv, b1.reshape(1, H), W2)
    agg = k["scatter"](y2, src_p, dst_p)
    y3 = k["tc_layer"](agg[0, :N], agg[1, :N], y2, dinv, b2.reshape(1, H), W3)
    agg = k["scatter"](y3, src_p, dst_p)
    P, Q = k["tc_final"](agg[0, :N], agg[1, :N], y3, dinv, b3.reshape(1, H),
                         Wc1[:H], Wc1[H:2 * H])
    Ps, Qd = k["gather"](P, Q, src_p, dst_g)
    out = k["tc_edge"](Ps[:E], Qd[:E], edge_attr,
                       We1, be1.reshape(1, H), We2, be2.reshape(1, H),
                       Wc1[2 * H:], bc1.reshape(1, H),
                       Wc2, bc2.reshape(1, H // 2), Wc3, bc3.reshape(1, 2))
    return out


# speedup vs baseline: 1.6777x; 1.1033x over previous
import functools

import jax
import jax.numpy as jnp
from jax import lax
from jax.experimental import pallas as pl
from jax.experimental.pallas import tpu as pltpu
from jax.experimental.pallas import tpu_sc as plsc

N = 10000
E = 320000
FN = 128
FE = 16
H = 64

NC = 2
NS = 16
NW = NC * NS
CHUNK = 128
NBUF = 4
K = (-(-E // (NW * CHUNK)) + NBUF - 1) // NBUF * NBUF
E_PAD = NW * K * CHUNK
N_ACC = 10240
RPT = N_ACC // NS
DUMMY = N


def _sc_mesh():
    return plsc.VectorSubcoreMesh(core_axis_name="c", subcore_axis_name="s",
                                  num_cores=NC, num_subcores=NS)


_SC_PARAMS = pltpu.CompilerParams(use_tc_tiling_on_sc=False)


def _zero_rows(ref, nrows, ncols):
    def body(i, _):
        for cth in range(ncols // 16):
            ref[i, pl.ds(cth * 16, 16)] = jnp.zeros((16,), jnp.float32)
        return 0
    lax.fori_loop(0, nrows, body, 0)


@functools.lru_cache(maxsize=None)
def _build_sc():
    interpret = False

    @functools.partial(
        pl.kernel,
        out_type=jax.ShapeDtypeStruct((NC, N_ACC, 16), jnp.float32),
        mesh=_sc_mesh(),
        scratch_types=[
            pltpu.VMEM((K, CHUNK), jnp.int32),
            pltpu.VMEM((CHUNK, 16), jnp.float32),
            pltpu.VMEM((RPT, 16), jnp.float32),
            pltpu.VMEM_SHARED((N_ACC, 16), jnp.float32),
            pltpu.SemaphoreType.DMA,
        ],
        compiler_params=_SC_PARAMS,
        interpret=interpret,
    )
    def deg_kernel(dst_hbm, out_hbm, dst_v, ones_v, stage_v, accum, sem):
        c = lax.axis_index("c")
        s = lax.axis_index("s")
        w = c * NS + s
        base = s * RPT

        def fill_ones(i, _):
            ones_v[i, :] = jnp.ones((16,), jnp.float32)
            return 0
        lax.fori_loop(0, CHUNK, fill_ones, 0)
        _zero_rows(stage_v, RPT, 16)

        pltpu.sync_copy(dst_hbm.at[w], dst_v)
        pltpu.sync_copy(stage_v, accum.at[pl.ds(base, RPT)])
        plsc.subcore_barrier()

        def body(j, _):
            pltpu.async_copy(ones_v, accum.at[dst_v.at[j]], sem, add=True)
            return 0
        lax.fori_loop(0, K, body, 0)

        def drain(j, _):
            pltpu.make_async_copy(ones_v, accum.at[dst_v.at[j]], sem).wait()
            return 0
        lax.fori_loop(0, K, drain, 0)

        plsc.subcore_barrier()
        pltpu.sync_copy(accum.at[pl.ds(base, RPT)], stage_v)
        pltpu.sync_copy(stage_v, out_hbm.at[c, pl.ds(base, RPT)])

    @functools.partial(
        pl.kernel,
        out_type=jax.ShapeDtypeStruct((NC, N_ACC, H), jnp.float32),
        mesh=_sc_mesh(),
        scratch_types=[
            pltpu.VMEM((K, CHUNK), jnp.int32),
            pltpu.VMEM((K, CHUNK), jnp.int32),
            pltpu.VMEM((NBUF, CHUNK, H), jnp.float32),
            pltpu.VMEM((CHUNK, H), jnp.float32),
            pltpu.VMEM_SHARED((N_ACC, H), jnp.float32),
            pltpu.SemaphoreType.DMA((NBUF,)),
            pltpu.SemaphoreType.DMA((NBUF,)),
        ],
        compiler_params=_SC_PARAMS,
        interpret=interpret,
    )
    def scatter_kernel(y_hbm, src_hbm, dst_hbm, out_hbm,
                       src_v, dst_v, rows_v, stage_v, accum, gsem, ssem):
        c = lax.axis_index("c")
        s = lax.axis_index("s")
        w = c * NS + s
        base = s * RPT

        _zero_rows(stage_v, CHUNK, H)
        pltpu.sync_copy(src_hbm.at[w], src_v)
        pltpu.sync_copy(dst_hbm.at[w], dst_v)
        for r in range(RPT // CHUNK):
            pltpu.sync_copy(stage_v, accum.at[pl.ds(base + r * CHUNK, CHUNK)])
        plsc.subcore_barrier()

        for b in range(NBUF):
            pltpu.async_copy(y_hbm.at[src_v.at[b]], rows_v.at[b], gsem.at[b])

        def body(jj, _):
            for b in range(NBUF):
                j = jj + b
                pltpu.make_async_copy(
                    y_hbm.at[src_v.at[j]], rows_v.at[b], gsem.at[b]).wait()
                pltpu.async_copy(
                    rows_v.at[b], accum.at[dst_v.at[j]], ssem.at[b], add=True)
            for b in range(NBUF):
                j = jj + b
                pltpu.make_async_copy(
                    rows_v.at[b], accum.at[dst_v.at[j]], ssem.at[b]).wait()

                @pl.when(j + NBUF < K)
                def _():
                    pltpu.async_copy(
                        y_hbm.at[src_v.at[j + NBUF]], rows_v.at[b], gsem.at[b])
            return 0
        lax.fori_loop(0, K // NBUF, lambda i, cc: body(i * NBUF, cc), 0)

        plsc.subcore_barrier()
        for r in range(RPT // CHUNK):
            pltpu.sync_copy(accum.at[pl.ds(base + r * CHUNK, CHUNK)], stage_v)
            pltpu.sync_copy(stage_v, out_hbm.at[c, pl.ds(base + r * CHUNK, CHUNK)])

    @functools.partial(
        pl.kernel,
        out_type=(jax.ShapeDtypeStruct((E_PAD, H), jnp.float32),
                  jax.ShapeDtypeStruct((E_PAD, H), jnp.float32)),
        mesh=_sc_mesh(),
        scratch_types=[
            pltpu.VMEM((K, CHUNK), jnp.int32),
            pltpu.VMEM((K, CHUNK), jnp.int32),
            pltpu.VMEM((NBUF, CHUNK, H), jnp.float32),
            pltpu.VMEM((NBUF, CHUNK, H), jnp.float32),
            pltpu.SemaphoreType.DMA((NBUF,)),
            pltpu.SemaphoreType.DMA((NBUF,)),
            pltpu.SemaphoreType.DMA((NBUF,)),
            pltpu.SemaphoreType.DMA((NBUF,)),
        ],
        compiler_params=_SC_PARAMS,
        interpret=interpret,
    )
    def gather_kernel(p_hbm, q_hbm, src_hbm, dst_hbm, outp_hbm, outq_hbm,
                      src_v, dst_v, bufp_v, bufq_v, gpsem, gqsem, wpsem, wqsem):
        c = lax.axis_index("c")
        s = lax.axis_index("s")
        w = c * NS + s
        ebase = w * K * CHUNK

        pltpu.sync_copy(src_hbm.at[w], src_v)
        pltpu.sync_copy(dst_hbm.at[w], dst_v)

        for b in range(NBUF):
            pltpu.async_copy(p_hbm.at[src_v.at[b]], bufp_v.at[b], gpsem.at[b])
            pltpu.async_copy(q_hbm.at[dst_v.at[b]], bufq_v.at[b], gqsem.at[b])

        def body(jj, _):
            for b in range(NBUF):
                j = jj + b
                off = ebase + j * CHUNK
                pltpu.make_async_copy(
                    p_hbm.at[src_v.at[j]], bufp_v.at[b], gpsem.at[b]).wait()
                pltpu.async_copy(
                    bufp_v.at[b], outp_hbm.at[pl.ds(off, CHUNK)], wpsem.at[b])
                pltpu.make_async_copy(
                    q_hbm.at[dst_v.at[j]], bufq_v.at[b], gqsem.at[b]).wait()
                pltpu.async_copy(
                    bufq_v.at[b], outq_hbm.at[pl.ds(off, CHUNK)], wqsem.at[b])
            for b in range(NBUF):
                j = jj + b
                off = ebase + j * CHUNK
                pltpu.make_async_copy(
                    bufp_v.at[b], outp_hbm.at[pl.ds(off, CHUNK)], wpsem.at[b]).wait()
                pltpu.make_async_copy(
                    bufq_v.at[b], outq_hbm.at[pl.ds(off, CHUNK)], wqsem.at[b]).wait()

                @pl.when(j + NBUF < K)
                def _():
                    pltpu.async_copy(
                        p_hbm.at[src_v.at[j + NBUF]], bufp_v.at[b], gpsem.at[b])
                    pltpu.async_copy(
                        q_hbm.at[dst_v.at[j + NBUF]], bufq_v.at[b], gqsem.at[b])
            return 0
        lax.fori_loop(0, K // NBUF, lambda i, cc: body(i * NBUF, cc), 0)

    return dict(deg=deg_kernel, scatter=scatter_kernel, gather=gather_kernel)


@functools.lru_cache(maxsize=None)
def _build_tc(interpret: bool = False):

    def _mm(a, b):
        return jnp.dot(a, b, preferred_element_type=jnp.float32)

    BLK_N = 1000
    BLK_E = 2560

    def tc_a_body(x_ref, w1_ref, d0_ref, d1_ref, y_ref, dinv_ref):
        dinv = lax.rsqrt(d0_ref[...] + d1_ref[...] + 1.0)
        y_ref[...] = _mm(x_ref[...], w1_ref[...]) * dinv
        dinv_ref[...] = dinv

    tc_a = pl.pallas_call(
        tc_a_body,
        grid=(N // BLK_N,),
        in_specs=[
            pl.BlockSpec((BLK_N, FN), lambda i: (i, 0)),
            pl.BlockSpec((FN, H), lambda i: (0, 0)),
            pl.BlockSpec((BLK_N, 1), lambda i: (i, 0)),
            pl.BlockSpec((BLK_N, 1), lambda i: (i, 0)),
        ],
        out_specs=[
            pl.BlockSpec((BLK_N, H), lambda i: (i, 0)),
            pl.BlockSpec((BLK_N, 1), lambda i: (i, 0)),
        ],
        out_shape=[
            jax.ShapeDtypeStruct((N, H), jnp.float32),
            jax.ShapeDtypeStruct((N, 1), jnp.float32),
        ],
        interpret=interpret,
    )

    def tc_layer_body(a0_ref, a1_ref, yp_ref, dinv_ref, b_ref, w_ref, y_ref):
        h = jnp.maximum(
            (a0_ref[...] + a1_ref[...] + yp_ref[...]) * dinv_ref[...] + b_ref[...],
            0.0)
        y_ref[...] = _mm(h, w_ref[...]) * dinv_ref[...]

    tc_layer = pl.pallas_call(
        tc_layer_body,
        grid=(N // BLK_N,),
        in_specs=[
            pl.BlockSpec((BLK_N, H), lambda i: (i, 0)),
            pl.BlockSpec((BLK_N, H), lambda i: (i, 0)),
            pl.BlockSpec((BLK_N, H), lambda i: (i, 0)),
            pl.BlockSpec((BLK_N, 1), lambda i: (i, 0)),
            pl.BlockSpec((1, H), lambda i: (0, 0)),
            pl.BlockSpec((H, H), lambda i: (0, 0)),
        ],
        out_specs=pl.BlockSpec((BLK_N, H), lambda i: (i, 0)),
        out_shape=jax.ShapeDtypeStruct((N, H), jnp.float32),
        interpret=interpret,
    )

    def tc_final_body(a0_ref, a1_ref, yp_ref, dinv_ref, b_ref, wa_ref, wb_ref,
                      p_ref, q_ref):
        h = jnp.maximum(
            (a0_ref[...] + a1_ref[...] + yp_ref[...]) * dinv_ref[...] + b_ref[...],
            0.0)
        p_ref[...] = _mm(h, wa_ref[...])
        q_ref[...] = _mm(h, wb_ref[...])

    tc_final = pl.pallas_call(
        tc_final_body,
        grid=(N // BLK_N,),
        in_specs=[
            pl.BlockSpec((BLK_N, H), lambda i: (i, 0)),
            pl.BlockSpec((BLK_N, H), lambda i: (i, 0)),
            pl.BlockSpec((BLK_N, H), lambda i: (i, 0)),
            pl.BlockSpec((BLK_N, 1), lambda i: (i, 0)),
            pl.BlockSpec((1, H), lambda i: (0, 0)),
            pl.BlockSpec((H, H), lambda i: (0, 0)),
            pl.BlockSpec((H, H), lambda i: (0, 0)),
        ],
        out_specs=[
            pl.BlockSpec((BLK_N, H), lambda i: (i, 0)),
            pl.BlockSpec((BLK_N, H), lambda i: (i, 0)),
        ],
        out_shape=[
            jax.ShapeDtypeStruct((N, H), jnp.float32),
            jax.ShapeDtypeStruct((N, H), jnp.float32),
        ],
        interpret=interpret,
    )

    def tc_edge_body(ps_ref, qd_ref, ea_ref, we1_ref, be1_ref, we2_ref, be2_ref,
                     wc1c_ref, bc1_ref, wc2_ref, bc2_ref, wc3_ref, bc3_ref,
                     out_ref):
        e1 = jnp.maximum(_mm(ea_ref[...], we1_ref[...]) + be1_ref[...], 0.0)
        wfold = _mm(we2_ref[...], wc1c_ref[...])
        bfold = _mm(be2_ref[...], wc1c_ref[...]) + bc1_ref[...]
        g = _mm(e1, wfold) + bfold
        z1 = jnp.maximum(ps_ref[...] + qd_ref[...] + g, 0.0)
        z2 = jnp.maximum(_mm(z1, wc2_ref[...]) + bc2_ref[...], 0.0)
        z3 = _mm(z2, wc3_ref[...]) + bc3_ref[...]
        m = jnp.max(z3, axis=1, keepdims=True)
        lse = m + jnp.log(jnp.sum(jnp.exp(z3 - m), axis=1, keepdims=True))
        out_ref[...] = z3 - lse

    LAST_EA = E // BLK_E - 1
    tc_edge = pl.pallas_call(
        tc_edge_body,
        grid=(E_PAD // BLK_E,),
        in_specs=[
            pl.BlockSpec((BLK_E, H), lambda i: (i, 0)),
            pl.BlockSpec((BLK_E, H), lambda i: (i, 0)),
            pl.BlockSpec((BLK_E, FE), lambda i: (jnp.minimum(i, LAST_EA), 0)),
            pl.BlockSpec((FE, H), lambda i: (0, 0)),
            pl.BlockSpec((1, H), lambda i: (0, 0)),
            pl.BlockSpec((H, H), lambda i: (0, 0)),
            pl.BlockSpec((1, H), lambda i: (0, 0)),
            pl.BlockSpec((H, H), lambda i: (0, 0)),
            pl.BlockSpec((1, H), lambda i: (0, 0)),
            pl.BlockSpec((H, H // 2), lambda i: (0, 0)),
            pl.BlockSpec((1, H // 2), lambda i: (0, 0)),
            pl.BlockSpec((H // 2, 2), lambda i: (0, 0)),
            pl.BlockSpec((1, 2), lambda i: (0, 0)),
        ],
        out_specs=pl.BlockSpec((BLK_E, 2), lambda i: (i, 0)),
        out_shape=jax.ShapeDtypeStruct((E_PAD, 2), jnp.float32),
        interpret=interpret,
    )

    return dict(tc_a=tc_a, tc_layer=tc_layer, tc_final=tc_final,
                tc_edge=tc_edge)


def kernel(x, edge_index, edge_attr, W1, b1, W2, b2, W3, b3,
           We1, be1, We2, be2, Wc1, bc1, Wc2, bc2, Wc3, bc3):
    k = dict(_build_sc())
    k.update(_build_tc(False))
    ei = edge_index.astype(jnp.int32)
    src, dst = ei[0], ei[1]
    pad = E_PAD - E
    ar = jnp.arange(pad, dtype=jnp.int32)
    pad_lo = ar % jnp.int32(256)
    pad_hi = DUMMY + ar % jnp.int32(N_ACC - N)
    def slab(v):
        return v.reshape(NW, K, CHUNK)
    src_p = slab(jnp.concatenate([src, pad_lo]))
    dst_p = slab(jnp.concatenate([dst, pad_hi]))
    dst_g = slab(jnp.concatenate([dst, pad_lo]))

    degp = k["deg"](dst_p)
    d0 = degp[0, :N, 0:1]
    d1 = degp[1, :N, 0:1]

    y1, dinv = k["tc_a"](x, W1, d0, d1)
    agg = k["scatter"](y1, src_p, dst_p)
    y2 = k["tc_layer"](agg[0, :N], agg[1, :N], y1, dinv, b1.reshape(1, H), W2)
    agg = k["scatter"](y2, src_p, dst_p)
    y3 = k["tc_layer"](agg[0, :N], agg[1, :N], y2, dinv, b2.reshape(1, H), W3)
    agg = k["scatter"](y3, src_p, dst_p)
    P, Q = k["tc_final"](agg[0, :N], agg[1, :N], y3, dinv, b3.reshape(1, H),
                         Wc1[:H], Wc1[H:2 * H])
    Ps, Qd = k["gather"](P, Q, src_p, dst_g)
    out = k["tc_edge"](Ps, Qd, edge_attr,
                       We1, be1.reshape(1, H), We2, be2.reshape(1, H),
                       Wc1[2 * H:], bc1.reshape(1, H),
                       Wc2, bc2.reshape(1, H // 2), Wc3, bc3.reshape(1, 2))
    return out[:E]


# speedup vs baseline: 2.2302x; 1.3294x over previous
import functools

import jax
import jax.numpy as jnp
from jax import lax
from jax.experimental import pallas as pl
from jax.experimental.pallas import tpu as pltpu
from jax.experimental.pallas import tpu_sc as plsc

N = 10000
E = 320000
FN = 128
FE = 16
H = 64

NC = 2
NS = 16
NW = NC * NS
CHUNK = 128
NBUF = 4
K = (-(-E // (NW * CHUNK)) + NBUF - 1) // NBUF * NBUF
E_PAD = NW * K * CHUNK
N_ACC = 10240
RPT = N_ACC // NS
DUMMY = N


def _sc_mesh():
    return plsc.VectorSubcoreMesh(core_axis_name="c", subcore_axis_name="s",
                                  num_cores=NC, num_subcores=NS)


_SC_PARAMS = pltpu.CompilerParams(use_tc_tiling_on_sc=False)


def _zero_rows(ref, nrows, ncols):
    def body(i, _):
        for cth in range(ncols // 16):
            ref[i, pl.ds(cth * 16, 16)] = jnp.zeros((16,), jnp.float32)
        return 0
    lax.fori_loop(0, nrows, body, 0)


@functools.lru_cache(maxsize=None)
def _build_sc():
    interpret = False

    @functools.partial(
        pl.kernel,
        out_type=jax.ShapeDtypeStruct((NC, N_ACC, 16), jnp.float32),
        mesh=_sc_mesh(),
        scratch_types=[
            pltpu.VMEM((K, CHUNK), jnp.int32),
            pltpu.VMEM((CHUNK, 16), jnp.float32),
            pltpu.VMEM((RPT, 16), jnp.float32),
            pltpu.VMEM_SHARED((N_ACC, 16), jnp.float32),
            pltpu.SemaphoreType.DMA,
        ],
        compiler_params=_SC_PARAMS,
        interpret=interpret,
    )
    def deg_kernel(dst_hbm, out_hbm, dst_v, ones_v, stage_v, accum, sem):
        c = lax.axis_index("c")
        s = lax.axis_index("s")
        w = c * NS + s
        base = s * RPT

        def fill_ones(i, _):
            ones_v[i, :] = jnp.ones((16,), jnp.float32)
            return 0
        lax.fori_loop(0, CHUNK, fill_ones, 0)
        _zero_rows(stage_v, RPT, 16)

        pltpu.sync_copy(dst_hbm.at[w], dst_v)
        pltpu.sync_copy(stage_v, accum.at[pl.ds(base, RPT)])
        plsc.subcore_barrier()

        def body(j, _):
            pltpu.async_copy(ones_v, accum.at[dst_v.at[j]], sem, add=True)
            return 0
        lax.fori_loop(0, K, body, 0)

        def drain(j, _):
            pltpu.make_async_copy(ones_v, accum.at[dst_v.at[j]], sem).wait()
            return 0
        lax.fori_loop(0, K, drain, 0)

        plsc.subcore_barrier()
        pltpu.sync_copy(accum.at[pl.ds(base, RPT)], stage_v)
        pltpu.sync_copy(stage_v, out_hbm.at[c, pl.ds(base, RPT)])

    @functools.partial(
        pl.kernel,
        out_type=jax.ShapeDtypeStruct((NC, N_ACC, H), jnp.float32),
        mesh=_sc_mesh(),
        scratch_types=[
            pltpu.VMEM((K, CHUNK), jnp.int32),
            pltpu.VMEM((K, CHUNK), jnp.int32),
            pltpu.VMEM((NBUF, CHUNK, H), jnp.float32),
            pltpu.VMEM((CHUNK, H), jnp.float32),
            pltpu.VMEM_SHARED((N_ACC, H), jnp.float32),
            pltpu.SemaphoreType.DMA((NBUF,)),
            pltpu.SemaphoreType.DMA((NBUF,)),
        ],
        compiler_params=_SC_PARAMS,
        interpret=interpret,
    )
    def scatter_kernel(y_hbm, src_hbm, dst_hbm, out_hbm,
                       src_v, dst_v, rows_v, stage_v, accum, gsem, ssem):
        c = lax.axis_index("c")
        s = lax.axis_index("s")
        w = c * NS + s
        base = s * RPT

        _zero_rows(stage_v, CHUNK, H)
        pltpu.sync_copy(src_hbm.at[w], src_v)
        pltpu.sync_copy(dst_hbm.at[w], dst_v)
        for r in range(RPT // CHUNK):
            pltpu.sync_copy(stage_v, accum.at[pl.ds(base + r * CHUNK, CHUNK)])
        plsc.subcore_barrier()

        for b in range(NBUF):
            pltpu.async_copy(y_hbm.at[src_v.at[b]], rows_v.at[b], gsem.at[b])

        def body(jj, _):
            for b in range(NBUF):
                j = jj + b
                pltpu.make_async_copy(
                    y_hbm.at[src_v.at[j]], rows_v.at[b], gsem.at[b]).wait()
                pltpu.async_copy(
                    rows_v.at[b], accum.at[dst_v.at[j]], ssem.at[b], add=True)
            for b in range(NBUF):
                j = jj + b
                pltpu.make_async_copy(
                    rows_v.at[b], accum.at[dst_v.at[j]], ssem.at[b]).wait()

                @pl.when(j + NBUF < K)
                def _():
                    pltpu.async_copy(
                        y_hbm.at[src_v.at[j + NBUF]], rows_v.at[b], gsem.at[b])
            return 0
        lax.fori_loop(0, K // NBUF, lambda i, cc: body(i * NBUF, cc), 0)

        plsc.subcore_barrier()
        for r in range(RPT // CHUNK):
            pltpu.sync_copy(accum.at[pl.ds(base + r * CHUNK, CHUNK)], stage_v)
            pltpu.sync_copy(stage_v, out_hbm.at[c, pl.ds(base + r * CHUNK, CHUNK)])

    GBUF = 2

    @functools.partial(
        pl.kernel,
        out_type=(jax.ShapeDtypeStruct((E_PAD, 2 * H), jnp.float32),
                  jax.ShapeDtypeStruct((E_PAD, 2 * H), jnp.float32)),
        mesh=_sc_mesh(),
        scratch_types=[
            pltpu.VMEM((K, CHUNK), jnp.int32),
            pltpu.VMEM((K, CHUNK), jnp.int32),
            pltpu.VMEM((GBUF, CHUNK, 2 * H), jnp.float32),
            pltpu.VMEM((GBUF, CHUNK, 2 * H), jnp.float32),
            pltpu.SemaphoreType.DMA((GBUF,)),
            pltpu.SemaphoreType.DMA((GBUF,)),
            pltpu.SemaphoreType.DMA((GBUF,)),
            pltpu.SemaphoreType.DMA((GBUF,)),
        ],
        compiler_params=_SC_PARAMS,
        interpret=interpret,
    )
    def gather_kernel(pq_hbm, src_hbm, dst_hbm, outp_hbm, outq_hbm,
                      src_v, dst_v, bufp_v, bufq_v, gpsem, gqsem, wpsem, wqsem):
        c = lax.axis_index("c")
        s = lax.axis_index("s")
        w = c * NS + s
        ebase = w * K * CHUNK

        pltpu.sync_copy(src_hbm.at[w], src_v)
        pltpu.sync_copy(dst_hbm.at[w], dst_v)

        for b in range(GBUF):
            pltpu.async_copy(pq_hbm.at[src_v.at[b]], bufp_v.at[b], gpsem.at[b])
            pltpu.async_copy(pq_hbm.at[dst_v.at[b]], bufq_v.at[b], gqsem.at[b])

        def body(jj, _):
            for b in range(GBUF):
                j = jj + b
                off = ebase + j * CHUNK
                pltpu.make_async_copy(
                    pq_hbm.at[src_v.at[j]], bufp_v.at[b], gpsem.at[b]).wait()
                pltpu.async_copy(
                    bufp_v.at[b], outp_hbm.at[pl.ds(off, CHUNK)], wpsem.at[b])
                pltpu.make_async_copy(
                    pq_hbm.at[dst_v.at[j]], bufq_v.at[b], gqsem.at[b]).wait()
                pltpu.async_copy(
                    bufq_v.at[b], outq_hbm.at[pl.ds(off, CHUNK)], wqsem.at[b])
            for b in range(GBUF):
                j = jj + b
                off = ebase + j * CHUNK
                pltpu.make_async_copy(
                    bufp_v.at[b], outp_hbm.at[pl.ds(off, CHUNK)], wpsem.at[b]).wait()
                pltpu.make_async_copy(
                    bufq_v.at[b], outq_hbm.at[pl.ds(off, CHUNK)], wqsem.at[b]).wait()

                @pl.when(j + GBUF < K)
                def _():
                    pltpu.async_copy(
                        pq_hbm.at[src_v.at[j + GBUF]], bufp_v.at[b], gpsem.at[b])
                    pltpu.async_copy(
                        pq_hbm.at[dst_v.at[j + GBUF]], bufq_v.at[b], gqsem.at[b])
            return 0
        lax.fori_loop(0, K // GBUF, lambda i, cc: body(i * GBUF, cc), 0)

    return dict(deg=deg_kernel, scatter=scatter_kernel, gather=gather_kernel)


@functools.lru_cache(maxsize=None)
def _build_tc(interpret: bool = False):

    def _mm(a, b):
        return jnp.dot(a, b, preferred_element_type=jnp.float32)

    BLK_N = 1000
    BLK_E = 2560

    def tc_a_body(x_ref, w1_ref, d0_ref, d1_ref, y_ref, dinv_ref):
        dinv = lax.rsqrt(d0_ref[...] + d1_ref[...] + 1.0)
        y_ref[...] = _mm(x_ref[...], w1_ref[...]) * dinv
        dinv_ref[...] = dinv

    tc_a = pl.pallas_call(
        tc_a_body,
        grid=(N // BLK_N,),
        in_specs=[
            pl.BlockSpec((BLK_N, FN), lambda i: (i, 0)),
            pl.BlockSpec((FN, H), lambda i: (0, 0)),
            pl.BlockSpec((BLK_N, 1), lambda i: (i, 0)),
            pl.BlockSpec((BLK_N, 1), lambda i: (i, 0)),
        ],
        out_specs=[
            pl.BlockSpec((BLK_N, H), lambda i: (i, 0)),
            pl.BlockSpec((BLK_N, 1), lambda i: (i, 0)),
        ],
        out_shape=[
            jax.ShapeDtypeStruct((N, H), jnp.float32),
            jax.ShapeDtypeStruct((N, 1), jnp.float32),
        ],
        interpret=interpret,
    )

    def tc_layer_body(a0_ref, a1_ref, yp_ref, dinv_ref, b_ref, w_ref, y_ref):
        h = jnp.maximum(
            (a0_ref[...] + a1_ref[...] + yp_ref[...]) * dinv_ref[...] + b_ref[...],
            0.0)
        y_ref[...] = _mm(h, w_ref[...]) * dinv_ref[...]

    tc_layer = pl.pallas_call(
        tc_layer_body,
        grid=(N // BLK_N,),
        in_specs=[
            pl.BlockSpec((BLK_N, H), lambda i: (i, 0)),
            pl.BlockSpec((BLK_N, H), lambda i: (i, 0)),
            pl.BlockSpec((BLK_N, H), lambda i: (i, 0)),
            pl.BlockSpec((BLK_N, 1), lambda i: (i, 0)),
            pl.BlockSpec((1, H), lambda i: (0, 0)),
            pl.BlockSpec((H, H), lambda i: (0, 0)),
        ],
        out_specs=pl.BlockSpec((BLK_N, H), lambda i: (i, 0)),
        out_shape=jax.ShapeDtypeStruct((N, H), jnp.float32),
        interpret=interpret,
    )

    def tc_final_body(a0_ref, a1_ref, yp_ref, dinv_ref, b_ref, wab_ref, pq_ref):
        h = jnp.maximum(
            (a0_ref[...] + a1_ref[...] + yp_ref[...]) * dinv_ref[...] + b_ref[...],
            0.0)
        pq_ref[...] = _mm(h, wab_ref[...])

    tc_final = pl.pallas_call(
        tc_final_body,
        grid=(N // BLK_N,),
        in_specs=[
            pl.BlockSpec((BLK_N, H), lambda i: (i, 0)),
            pl.BlockSpec((BLK_N, H), lambda i: (i, 0)),
            pl.BlockSpec((BLK_N, H), lambda i: (i, 0)),
            pl.BlockSpec((BLK_N, 1), lambda i: (i, 0)),
            pl.BlockSpec((1, H), lambda i: (0, 0)),
            pl.BlockSpec((H, 2 * H), lambda i: (0, 0)),
        ],
        out_specs=pl.BlockSpec((BLK_N, 2 * H), lambda i: (i, 0)),
        out_shape=jax.ShapeDtypeStruct((N, 2 * H), jnp.float32),
        interpret=interpret,
    )

    def tc_edge_body(ps_ref, qd_ref, ea_ref, we1_ref, be1_ref, we2_ref, be2_ref,
                     wc1c_ref, bc1_ref, wc2_ref, bc2_ref, wc3_ref, bc3_ref,
                     out_ref):
        e1 = jnp.maximum(_mm(ea_ref[...], we1_ref[...]) + be1_ref[...], 0.0)
        wfold = _mm(we2_ref[...], wc1c_ref[...])
        bfold = _mm(be2_ref[...], wc1c_ref[...]) + bc1_ref[...]
        g = _mm(e1, wfold) + bfold
        z1 = jnp.maximum(ps_ref[:, :H] + qd_ref[:, H:] + g, 0.0)
        z2 = jnp.maximum(_mm(z1, wc2_ref[...]) + bc2_ref[...], 0.0)
        z3 = _mm(z2, wc3_ref[...]) + bc3_ref[...]
        m = jnp.max(z3, axis=1, keepdims=True)
        lse = m + jnp.log(jnp.sum(jnp.exp(z3 - m), axis=1, keepdims=True))
        out_ref[...] = jnp.swapaxes(z3 - lse, 0, 1)

    LAST_EA = E // BLK_E - 1
    tc_edge = pl.pallas_call(
        tc_edge_body,
        grid=(E_PAD // BLK_E,),
        in_specs=[
            pl.BlockSpec((BLK_E, 2 * H), lambda i: (i, 0)),
            pl.BlockSpec((BLK_E, 2 * H), lambda i: (i, 0)),
            pl.BlockSpec((BLK_E, FE), lambda i: (jnp.minimum(i, LAST_EA), 0)),
            pl.BlockSpec((FE, H), lambda i: (0, 0)),
            pl.BlockSpec((1, H), lambda i: (0, 0)),
            pl.BlockSpec((H, H), lambda i: (0, 0)),
            pl.BlockSpec((1, H), lambda i: (0, 0)),
            pl.BlockSpec((H, H), lambda i: (0, 0)),
            pl.BlockSpec((1, H), lambda i: (0, 0)),
            pl.BlockSpec((H, H // 2), lambda i: (0, 0)),
            pl.BlockSpec((1, H // 2), lambda i: (0, 0)),
            pl.BlockSpec((H // 2, 2), lambda i: (0, 0)),
            pl.BlockSpec((1, 2), lambda i: (0, 0)),
        ],
        out_specs=pl.BlockSpec((2, BLK_E), lambda i: (0, i)),
        out_shape=jax.ShapeDtypeStruct((2, E_PAD), jnp.float32),
        interpret=interpret,
    )

    return dict(tc_a=tc_a, tc_layer=tc_layer, tc_final=tc_final,
                tc_edge=tc_edge)


def kernel(x, edge_index, edge_attr, W1, b1, W2, b2, W3, b3,
           We1, be1, We2, be2, Wc1, bc1, Wc2, bc2, Wc3, bc3):
    k = dict(_build_sc())
    k.update(_build_tc(False))
    ei = edge_index.astype(jnp.int32)
    src, dst = ei[0], ei[1]
    pad = E_PAD - E
    ar = jnp.arange(pad, dtype=jnp.int32)
    pad_lo = ar % jnp.int32(256)
    pad_hi = DUMMY + ar % jnp.int32(N_ACC - N)
    def slab(v):
        return v.reshape(NW, K, CHUNK)
    src_p = slab(jnp.concatenate([src, pad_lo]))
    dst_p = slab(jnp.concatenate([dst, pad_hi]))
    dst_g = slab(jnp.concatenate([dst, pad_lo]))

    degp = k["deg"](dst_p)
    d0 = degp[0, :N, 0:1]
    d1 = degp[1, :N, 0:1]

    y1, dinv = k["tc_a"](x, W1, d0, d1)
    agg = k["scatter"](y1, src_p, dst_p)
    y2 = k["tc_layer"](agg[0, :N], agg[1, :N], y1, dinv, b1.reshape(1, H), W2)
    agg = k["scatter"](y2, src_p, dst_p)
    y3 = k["tc_layer"](agg[0, :N], agg[1, :N], y2, dinv, b2.reshape(1, H), W3)
    agg = k["scatter"](y3, src_p, dst_p)
    Wab = jnp.concatenate([Wc1[:H], Wc1[H:2 * H]], axis=1)
    PQ = k["tc_final"](agg[0, :N], agg[1, :N], y3, dinv, b3.reshape(1, H), Wab)
    Ps, Qd = k["gather"](PQ, src_p, dst_g)
    out = k["tc_edge"](Ps, Qd, edge_attr,
                       We1, be1.reshape(1, H), We2, be2.reshape(1, H),
                       Wc1[2 * H:], bc1.reshape(1, H),
                       Wc2, bc2.reshape(1, H // 2), Wc3, bc3.reshape(1, 2))
    return out[:, :E].T


# speedup vs baseline: 2.3147x; 1.0379x over previous
import functools

import jax
import jax.numpy as jnp
from jax import lax
from jax.experimental import pallas as pl
from jax.experimental.pallas import tpu as pltpu
from jax.experimental.pallas import tpu_sc as plsc

N = 10000
E = 320000
FN = 128
FE = 16
H = 64

NC = 2
NS = 16
NW = NC * NS
CHUNK = 128
NBUF = 4
K = (-(-E // (NW * CHUNK)) + NBUF - 1) // NBUF * NBUF
E_PAD = NW * K * CHUNK
N_ACC = 10240
RPT = N_ACC // NS
DUMMY = N


def _sc_mesh():
    return plsc.VectorSubcoreMesh(core_axis_name="c", subcore_axis_name="s",
                                  num_cores=NC, num_subcores=NS)


_SC_PARAMS = pltpu.CompilerParams(use_tc_tiling_on_sc=False)


def _zero_rows(ref, nrows, ncols):
    def body(i, _):
        for cth in range(ncols // 16):
            ref[i, pl.ds(cth * 16, 16)] = jnp.zeros((16,), jnp.float32)
        return 0
    lax.fori_loop(0, nrows, body, 0)


@functools.lru_cache(maxsize=None)
def _build_sc():
    interpret = False

    @functools.partial(
        pl.kernel,
        out_type=jax.ShapeDtypeStruct((NC, N_ACC, 16), jnp.float32),
        mesh=_sc_mesh(),
        scratch_types=[
            pltpu.VMEM((K, CHUNK), jnp.int32),
            pltpu.VMEM((CHUNK, 16), jnp.float32),
            pltpu.VMEM((RPT, 16), jnp.float32),
            pltpu.VMEM_SHARED((N_ACC, 16), jnp.float32),
            pltpu.SemaphoreType.DMA,
        ],
        compiler_params=_SC_PARAMS,
        interpret=interpret,
    )
    def deg_kernel(dst_hbm, out_hbm, dst_v, ones_v, stage_v, accum, sem):
        c = lax.axis_index("c")
        s = lax.axis_index("s")
        w = c * NS + s
        base = s * RPT

        def fill_ones(i, _):
            ones_v[i, :] = jnp.ones((16,), jnp.float32)
            return 0
        lax.fori_loop(0, CHUNK, fill_ones, 0)
        _zero_rows(stage_v, RPT, 16)

        pltpu.sync_copy(dst_hbm.at[w], dst_v)
        pltpu.sync_copy(stage_v, accum.at[pl.ds(base, RPT)])
        plsc.subcore_barrier()

        def body(j, _):
            pltpu.async_copy(ones_v, accum.at[dst_v.at[j]], sem, add=True)
            return 0
        lax.fori_loop(0, K, body, 0)

        def drain(j, _):
            pltpu.make_async_copy(ones_v, accum.at[dst_v.at[j]], sem).wait()
            return 0
        lax.fori_loop(0, K, drain, 0)

        plsc.subcore_barrier()
        pltpu.sync_copy(accum.at[pl.ds(base, RPT)], stage_v)
        pltpu.sync_copy(stage_v, out_hbm.at[c, pl.ds(base, RPT)])

    @functools.partial(
        pl.kernel,
        out_type=jax.ShapeDtypeStruct((NC, N_ACC, H), jnp.float32),
        mesh=_sc_mesh(),
        scratch_types=[
            pltpu.VMEM((K, CHUNK), jnp.int32),
            pltpu.VMEM((K, CHUNK), jnp.int32),
            pltpu.VMEM((NBUF, CHUNK, H), jnp.float32),
            pltpu.VMEM((CHUNK, H), jnp.float32),
            pltpu.VMEM_SHARED((N_ACC, H), jnp.float32),
            pltpu.SemaphoreType.DMA((NBUF,)),
            pltpu.SemaphoreType.DMA((NBUF,)),
        ],
        compiler_params=_SC_PARAMS,
        interpret=interpret,
    )
    def scatter_kernel(y_hbm, src_hbm, dst_hbm, out_hbm,
                       src_v, dst_v, rows_v, stage_v, accum, gsem, ssem):
        c = lax.axis_index("c")
        s = lax.axis_index("s")
        w = c * NS + s
        base = s * RPT

        _zero_rows(stage_v, CHUNK, H)
        pltpu.sync_copy(src_hbm.at[w], src_v)
        pltpu.sync_copy(dst_hbm.at[w], dst_v)
        for r in range(RPT // CHUNK):
            pltpu.sync_copy(stage_v, accum.at[pl.ds(base + r * CHUNK, CHUNK)])
        plsc.subcore_barrier()

        for b in range(NBUF):
            pltpu.async_copy(y_hbm.at[src_v.at[b]], rows_v.at[b], gsem.at[b])

        def body(jj, _):
            for b in range(NBUF):
                j = jj + b
                pltpu.make_async_copy(
                    y_hbm.at[src_v.at[j]], rows_v.at[b], gsem.at[b]).wait()
                pltpu.async_copy(
                    rows_v.at[b], accum.at[dst_v.at[j]], ssem.at[b], add=True)
            for b in range(NBUF):
                j = jj + b
                pltpu.make_async_copy(
                    rows_v.at[b], accum.at[dst_v.at[j]], ssem.at[b]).wait()

                @pl.when(j + NBUF < K)
                def _():
                    pltpu.async_copy(
                        y_hbm.at[src_v.at[j + NBUF]], rows_v.at[b], gsem.at[b])
            return 0
        lax.fori_loop(0, K // NBUF, lambda i, cc: body(i * NBUF, cc), 0)

        plsc.subcore_barrier()
        for r in range(RPT // CHUNK):
            pltpu.sync_copy(accum.at[pl.ds(base + r * CHUNK, CHUNK)], stage_v)
            pltpu.sync_copy(stage_v, out_hbm.at[c, pl.ds(base + r * CHUNK, CHUNK)])

    GBUF = 2

    @functools.partial(
        pl.kernel,
        out_type=jax.ShapeDtypeStruct((E_PAD, 2 * H), jnp.float32),
        mesh=_sc_mesh(),
        scratch_types=[
            pltpu.VMEM((K, CHUNK), jnp.int32),
            pltpu.VMEM((K, CHUNK), jnp.int32),
            pltpu.VMEM((GBUF, CHUNK, H), jnp.float32),
            pltpu.VMEM((GBUF, CHUNK, H), jnp.float32),
            pltpu.VMEM((GBUF, CHUNK, 2 * H), jnp.float32),
            pltpu.SemaphoreType.DMA((GBUF,)),
            pltpu.SemaphoreType.DMA((GBUF,)),
            pltpu.SemaphoreType.DMA((GBUF,)),
        ],
        compiler_params=_SC_PARAMS,
        interpret=interpret,
    )
    def gather_kernel(p_hbm, q_hbm, src_hbm, dst_hbm, outpq_hbm,
                      src_v, dst_v, bufp_v, bufq_v, outbuf_v, gpsem, gqsem, wsem):
        c = lax.axis_index("c")
        s = lax.axis_index("s")
        w = c * NS + s
        ebase = w * K * CHUNK

        pltpu.sync_copy(src_hbm.at[w], src_v)
        pltpu.sync_copy(dst_hbm.at[w], dst_v)

        for b in range(GBUF):
            pltpu.async_copy(p_hbm.at[src_v.at[b]], bufp_v.at[b], gpsem.at[b])
            pltpu.async_copy(q_hbm.at[dst_v.at[b]], bufq_v.at[b], gqsem.at[b])

        def body(jj, _):
            for b in range(GBUF):
                j = jj + b
                off = ebase + j * CHUNK

                @pl.when(j >= GBUF)
                def _():
                    pltpu.make_async_copy(
                        outbuf_v.at[b], outpq_hbm.at[pl.ds(ebase, CHUNK)],
                        wsem.at[b]).wait()

                pltpu.make_async_copy(
                    p_hbm.at[src_v.at[j]], bufp_v.at[b], gpsem.at[b]).wait()
                pltpu.make_async_copy(
                    q_hbm.at[dst_v.at[j]], bufq_v.at[b], gqsem.at[b]).wait()

                def asm(r, _):
                    for c4 in range(H // 16):
                        outbuf_v[b, r, pl.ds(c4 * 16, 16)] = \
                            bufp_v[b, r, pl.ds(c4 * 16, 16)]
                        outbuf_v[b, r, pl.ds(H + c4 * 16, 16)] = \
                            bufq_v[b, r, pl.ds(c4 * 16, 16)]
                    return 0
                lax.fori_loop(0, CHUNK, asm, 0)

                pltpu.async_copy(
                    outbuf_v.at[b], outpq_hbm.at[pl.ds(off, CHUNK)], wsem.at[b])

                @pl.when(j + GBUF < K)
                def _():
                    pltpu.async_copy(
                        p_hbm.at[src_v.at[j + GBUF]], bufp_v.at[b], gpsem.at[b])
                    pltpu.async_copy(
                        q_hbm.at[dst_v.at[j + GBUF]], bufq_v.at[b], gqsem.at[b])
            return 0
        lax.fori_loop(0, K // GBUF, lambda i, cc: body(i * GBUF, cc), 0)

        for b in range(GBUF):
            pltpu.make_async_copy(
                outbuf_v.at[b], outpq_hbm.at[pl.ds(ebase, CHUNK)], wsem.at[b]).wait()

    return dict(deg=deg_kernel, scatter=scatter_kernel, gather=gather_kernel)


@functools.lru_cache(maxsize=None)
def _build_tc(interpret: bool = False):

    def _mm(a, b):
        return jnp.dot(a, b, preferred_element_type=jnp.float32)

    BLK_N = 1000
    BLK_E = 2560

    def tc_a_body(x_ref, w1_ref, d0_ref, d1_ref, y_ref, dinv_ref):
        dinv = lax.rsqrt(d0_ref[...] + d1_ref[...] + 1.0)
        y_ref[...] = _mm(x_ref[...], w1_ref[...]) * dinv
        dinv_ref[...] = dinv

    tc_a = pl.pallas_call(
        tc_a_body,
        grid=(N // BLK_N,),
        in_specs=[
            pl.BlockSpec((BLK_N, FN), lambda i: (i, 0)),
            pl.BlockSpec((FN, H), lambda i: (0, 0)),
            pl.BlockSpec((BLK_N, 1), lambda i: (i, 0)),
            pl.BlockSpec((BLK_N, 1), lambda i: (i, 0)),
        ],
        out_specs=[
            pl.BlockSpec((BLK_N, H), lambda i: (i, 0)),
            pl.BlockSpec((BLK_N, 1), lambda i: (i, 0)),
        ],
        out_shape=[
            jax.ShapeDtypeStruct((N, H), jnp.float32),
            jax.ShapeDtypeStruct((N, 1), jnp.float32),
        ],
        interpret=interpret,
    )

    def tc_layer_body(a0_ref, a1_ref, yp_ref, dinv_ref, b_ref, w_ref, y_ref):
        h = jnp.maximum(
            (a0_ref[...] + a1_ref[...] + yp_ref[...]) * dinv_ref[...] + b_ref[...],
            0.0)
        y_ref[...] = _mm(h, w_ref[...]) * dinv_ref[...]

    tc_layer = pl.pallas_call(
        tc_layer_body,
        grid=(N // BLK_N,),
        in_specs=[
            pl.BlockSpec((BLK_N, H), lambda i: (i, 0)),
            pl.BlockSpec((BLK_N, H), lambda i: (i, 0)),
            pl.BlockSpec((BLK_N, H), lambda i: (i, 0)),
            pl.BlockSpec((BLK_N, 1), lambda i: (i, 0)),
            pl.BlockSpec((1, H), lambda i: (0, 0)),
            pl.BlockSpec((H, H), lambda i: (0, 0)),
        ],
        out_specs=pl.BlockSpec((BLK_N, H), lambda i: (i, 0)),
        out_shape=jax.ShapeDtypeStruct((N, H), jnp.float32),
        interpret=interpret,
    )

    def tc_final_body(a0_ref, a1_ref, yp_ref, dinv_ref, b_ref, wa_ref, wb_ref,
                      p_ref, q_ref):
        h = jnp.maximum(
            (a0_ref[...] + a1_ref[...] + yp_ref[...]) * dinv_ref[...] + b_ref[...],
            0.0)
        p_ref[...] = _mm(h, wa_ref[...])
        q_ref[...] = _mm(h, wb_ref[...])

    tc_final = pl.pallas_call(
        tc_final_body,
        grid=(N // BLK_N,),
        in_specs=[
            pl.BlockSpec((BLK_N, H), lambda i: (i, 0)),
            pl.BlockSpec((BLK_N, H), lambda i: (i, 0)),
            pl.BlockSpec((BLK_N, H), lambda i: (i, 0)),
            pl.BlockSpec((BLK_N, 1), lambda i: (i, 0)),
            pl.BlockSpec((1, H), lambda i: (0, 0)),
            pl.BlockSpec((H, H), lambda i: (0, 0)),
            pl.BlockSpec((H, H), lambda i: (0, 0)),
        ],
        out_specs=[
            pl.BlockSpec((BLK_N, H), lambda i: (i, 0)),
            pl.BlockSpec((BLK_N, H), lambda i: (i, 0)),
        ],
        out_shape=[
            jax.ShapeDtypeStruct((N, H), jnp.float32),
            jax.ShapeDtypeStruct((N, H), jnp.float32),
        ],
        interpret=interpret,
    )

    def tc_edge_body(pq_ref, ea_ref, we1_ref, be1_ref, we2_ref, be2_ref,
                     wc1c_ref, bc1_ref, wc2_ref, bc2_ref, wc3_ref, bc3_ref,
                     out_ref):
        e1 = jnp.maximum(_mm(ea_ref[...], we1_ref[...]) + be1_ref[...], 0.0)
        wfold = _mm(we2_ref[...], wc1c_ref[...])
        bfold = _mm(be2_ref[...], wc1c_ref[...]) + bc1_ref[...]
        g = _mm(e1, wfold) + bfold
        z1 = jnp.maximum(pq_ref[:, :H] + pq_ref[:, H:] + g, 0.0)
        z2 = jnp.maximum(_mm(z1, wc2_ref[...]) + bc2_ref[...], 0.0)
        z3 = _mm(z2, wc3_ref[...]) + bc3_ref[...]
        m = jnp.max(z3, axis=1, keepdims=True)
        lse = m + jnp.log(jnp.sum(jnp.exp(z3 - m), axis=1, keepdims=True))
        out_ref[...] = jnp.swapaxes(z3 - lse, 0, 1)

    LAST_EA = E // BLK_E - 1
    tc_edge = pl.pallas_call(
        tc_edge_body,
        grid=(E_PAD // BLK_E,),
        in_specs=[
            pl.BlockSpec((BLK_E, 2 * H), lambda i: (i, 0)),
            pl.BlockSpec((BLK_E, FE), lambda i: (jnp.minimum(i, LAST_EA), 0)),
            pl.BlockSpec((FE, H), lambda i: (0, 0)),
            pl.BlockSpec((1, H), lambda i: (0, 0)),
            pl.BlockSpec((H, H), lambda i: (0, 0)),
            pl.BlockSpec((1, H), lambda i: (0, 0)),
            pl.BlockSpec((H, H), lambda i: (0, 0)),
            pl.BlockSpec((1, H), lambda i: (0, 0)),
            pl.BlockSpec((H, H // 2), lambda i: (0, 0)),
            pl.BlockSpec((1, H // 2), lambda i: (0, 0)),
            pl.BlockSpec((H // 2, 2), lambda i: (0, 0)),
            pl.BlockSpec((1, 2), lambda i: (0, 0)),
        ],
        out_specs=pl.BlockSpec((2, BLK_E), lambda i: (0, i)),
        out_shape=jax.ShapeDtypeStruct((2, E_PAD), jnp.float32),
        interpret=interpret,
    )

    return dict(tc_a=tc_a, tc_layer=tc_layer, tc_final=tc_final,
                tc_edge=tc_edge)


def kernel(x, edge_index, edge_attr, W1, b1, W2, b2, W3, b3,
           We1, be1, We2, be2, Wc1, bc1, Wc2, bc2, Wc3, bc3):
    k = dict(_build_sc())
    k.update(_build_tc(False))
    ei = edge_index.astype(jnp.int32)
    src, dst = ei[0], ei[1]
    pad = E_PAD - E
    ar = jnp.arange(pad, dtype=jnp.int32)
    pad_lo = ar % jnp.int32(256)
    pad_hi = DUMMY + ar % jnp.int32(N_ACC - N)
    def slab(v):
        return v.reshape(NW, K, CHUNK)
    src_p = slab(jnp.concatenate([src, pad_lo]))
    dst_p = slab(jnp.concatenate([dst, pad_hi]))
    dst_g = slab(jnp.concatenate([dst, pad_lo]))

    degp = k["deg"](dst_p)
    d0 = degp[0, :N, 0:1]
    d1 = degp[1, :N, 0:1]

    y1, dinv = k["tc_a"](x, W1, d0, d1)
    agg = k["scatter"](y1, src_p, dst_p)
    y2 = k["tc_layer"](agg[0, :N], agg[1, :N], y1, dinv, b1.reshape(1, H), W2)
    agg = k["scatter"](y2, src_p, dst_p)
    y3 = k["tc_layer"](agg[0, :N], agg[1, :N], y2, dinv, b2.reshape(1, H), W3)
    agg = k["scatter"](y3, src_p, dst_p)
    P, Q = k["tc_final"](agg[0, :N], agg[1, :N], y3, dinv, b3.reshape(1, H),
                         Wc1[:H], Wc1[H:2 * H])
    Pq = k["gather"](P, Q, src_p, dst_g)
    out = k["tc_edge"](Pq, edge_attr,
                       We1, be1.reshape(1, H), We2, be2.reshape(1, H),
                       Wc1[2 * H:], bc1.reshape(1, H),
                       Wc2, bc2.reshape(1, H // 2), Wc3, bc3.reshape(1, 2))
    return out[:, :E].T
